# baseline (speedup 1.0000x reference)
import functools

import numpy as np
import jax
import jax.numpy as jnp
from jax import lax
from jax.experimental import pallas as pl
from jax.experimental.pallas import tpu as pltpu
from jax.experimental.pallas import tpu_sc as plsc

NS = 8
TILE = 256
ROWPAD = 16


def _fuse_weights(W1, b1, W2, b2):
    mlp1 = W2.shape[0]
    hi = jax.lax.Precision.HIGHEST
    terms = []
    for u in range(2):
        for v in range(2):
            t = jnp.einsum('om,mcij->ocij', W2[:, :, u, v], W1, precision=hi)
            terms.append(jnp.pad(t, ((0, 0), (0, 0), (u, 1 - u), (v, 1 - v))))
    K = terms[0] + terms[1] + terms[2] + terms[3]
    bias = b2 + jnp.einsum('omuv,m->o', W2, b1, precision=hi)
    K4 = K.reshape(mlp1, 4, 6, 4, 4)
    P = {(0, 0): K4[:, 0], (1, 1): K4[:, 1], (0, 1): K4[:, 2], (1, 0): K4[:, 3]}
    rows = []
    for dy in range(2):
        r = jnp.stack([P[(dy, 0)], P[(dy, 1)]], axis=-1)
        rows.append(r.reshape(mlp1, 6, 4, 8))
    A = jnp.stack(rows, axis=-2).reshape(mlp1, 6, 8, 8)
    cvec = bias + jnp.sum(A[:, 3:6], axis=(1, 2, 3))
    Wq = A[:, 0:3].reshape(mlp1, 3 * NS * NS).T
    return Wq, cvec


def _selection_mats():
    Sa = np.zeros((NS * ROWPAD, 3 * NS * NS), np.float32)
    Sb = np.zeros((NS * ROWPAD, 3 * NS * NS), np.float32)
    for c in range(3):
        for i in range(NS):
            for j in range(NS):
                k = c * NS * NS + i * NS + j
                Sa[i * ROWPAD + c, k] = 1.0
                Sb[j * ROWPAD + c, k] = 1.0
    return jnp.asarray(Sa), jnp.asarray(Sb)


def _topk_body(posT_tile_ref, posT_full_ref, pos3_ref, idx_ref):
    b = pl.program_id(0)
    n_full = pos3_ref.shape[2]
    posT_t = posT_tile_ref[0]
    pos3 = pos3_ref[0]
    sq_row = jnp.sum(pos3 * pos3, axis=0, keepdims=True)
    sq_col = jnp.sum(posT_t * posT_t, axis=1, keepdims=True)
    dot = jnp.dot(posT_t, pos3, preferred_element_type=jnp.float32)
    d = (sq_col + sq_row) - 2.0 * dot
    col = lax.broadcasted_iota(jnp.int32, d.shape, 1)
    work = d
    cols = []
    for _ in range(NS):
        idx_s = jnp.argmin(work, axis=1).astype(jnp.int32)[:, None]
        cols.append(idx_s)
        work = jnp.where(col == idx_s, jnp.float32(jnp.inf), work)
    idx_ref[...] = jnp.concatenate(cols, axis=1) + b * n_full


def _feat_body(g_ref, sa_ref, sb_ref, wq_ref, cvec_ref, out_ref):
    hi = jax.lax.Precision.HIGHEST
    G = g_ref[...]
    fa = jnp.dot(G, sa_ref[...], precision=hi,
                 preferred_element_type=jnp.float32)
    fb = jnp.dot(G, sb_ref[...], precision=hi,
                 preferred_element_type=jnp.float32)
    f = fa * fb
    out = jnp.dot(f, wq_ref[...], precision=hi,
                  preferred_element_type=jnp.float32)
    out = out + cvec_ref[...]
    out_ref[0] = out.T


def _sc_gather(table, idx_flat):
    n_idx = idx_flat.shape[0]
    info = plsc.get_sparse_core_info()
    nw = info.num_cores * info.num_subcores
    bpw = n_idx // nw
    mesh = plsc.VectorSubcoreMesh(core_axis_name="c", subcore_axis_name="s")

    @functools.partial(
        pl.kernel,
        out_type=jax.ShapeDtypeStruct((n_idx, ROWPAD), jnp.float32),
        mesh=mesh,
        compiler_params=pltpu.CompilerParams(use_tc_tiling_on_sc=False),
        scratch_types=[
            pltpu.VMEM((bpw,), jnp.int32),
            pltpu.VMEM((bpw, ROWPAD), jnp.float32),
            pltpu.SemaphoreType.DMA,
        ],
    )
    def gather_k(table_hbm, idx_hbm, out_hbm, idx_v, rows_v, sem):
        wid = lax.axis_index("s") * info.num_cores + lax.axis_index("c")
        base = wid * bpw
        pltpu.sync_copy(idx_hbm.at[pl.ds(base, bpw)], idx_v)
        pltpu.async_copy(table_hbm.at[idx_v], rows_v, sem).wait()
        pltpu.sync_copy(rows_v, out_hbm.at[pl.ds(base, bpw)])

    return gather_k(table, idx_flat)


def kernel(pos, W1, b1, W2, b2):
    B, C, N = pos.shape
    nt = N // TILE
    pos_t = jnp.transpose(pos, (0, 2, 1))

    idx = pl.pallas_call(
        _topk_body,
        grid=(B, nt),
        in_specs=[
            pl.BlockSpec((1, TILE, C), lambda b, t: (b, t, 0)),
            pl.BlockSpec((1, N, C), lambda b, t: (b, 0, 0)),
            pl.BlockSpec((1, C, N), lambda b, t: (b, 0, 0)),
        ],
        out_specs=pl.BlockSpec((TILE, NS), lambda b, t: (b * nt + t, 0)),
        out_shape=jax.ShapeDtypeStruct((B * N, NS), jnp.int32),
    )(pos_t, pos_t, pos)

    table = jnp.pad(pos_t.reshape(B * N, C), ((0, 0), (0, ROWPAD - C)))
    rows = _sc_gather(table, idx.reshape(B * N * NS))
    G = rows.reshape(B * N, NS * ROWPAD)

    Wq, cvec = _fuse_weights(W1, b1, W2, b2)
    Sa, Sb = _selection_mats()
    mlp1 = W2.shape[0]

    out = pl.pallas_call(
        _feat_body,
        grid=(B, nt),
        in_specs=[
            pl.BlockSpec((TILE, NS * ROWPAD), lambda b, t: (b * nt + t, 0)),
            pl.BlockSpec((NS * ROWPAD, 3 * NS * NS), lambda b, t: (0, 0)),
            pl.BlockSpec((NS * ROWPAD, 3 * NS * NS), lambda b, t: (0, 0)),
            pl.BlockSpec((3 * NS * NS, mlp1), lambda b, t: (0, 0)),
            pl.BlockSpec((1, mlp1), lambda b, t: (0, 0)),
        ],
        out_specs=pl.BlockSpec((1, mlp1, TILE), lambda b, t: (b, 0, t)),
        out_shape=jax.ShapeDtypeStruct((B, mlp1, N), jnp.float32),
    )(G, Sa, Sb, Wq, cvec[None, :])
    return out


# speedup vs baseline: 1.2379x; 1.2379x over previous
import functools

import numpy as np
import jax
import jax.numpy as jnp
from jax import lax
from jax.experimental import pallas as pl
from jax.experimental.pallas import tpu as pltpu
from jax.experimental.pallas import tpu_sc as plsc

NS = 8
TILE = 1024
ROWPAD = 16


def _fuse_weights(W1, b1, W2, b2):
    mlp1 = W2.shape[0]
    hi = jax.lax.Precision.HIGHEST
    terms = []
    for u in range(2):
        for v in range(2):
            t = jnp.einsum('om,mcij->ocij', W2[:, :, u, v], W1, precision=hi)
            terms.append(jnp.pad(t, ((0, 0), (0, 0), (u, 1 - u), (v, 1 - v))))
    K = terms[0] + terms[1] + terms[2] + terms[3]
    bias = b2 + jnp.einsum('omuv,m->o', W2, b1, precision=hi)
    K4 = K.reshape(mlp1, 4, 6, 4, 4)
    P = {(0, 0): K4[:, 0], (1, 1): K4[:, 1], (0, 1): K4[:, 2], (1, 0): K4[:, 3]}
    rows = []
    for dy in range(2):
        r = jnp.stack([P[(dy, 0)], P[(dy, 1)]], axis=-1)
        rows.append(r.reshape(mlp1, 6, 4, 8))
    A = jnp.stack(rows, axis=-2).reshape(mlp1, 6, 8, 8)
    cvec = bias + jnp.sum(A[:, 3:6], axis=(1, 2, 3))
    Wq = A[:, 0:3].reshape(mlp1, 3 * NS * NS).T
    return Wq, cvec


def _selection_idx():
    a = np.zeros((3 * NS * NS,), np.int32)
    b = np.zeros((3 * NS * NS,), np.int32)
    for c in range(3):
        for i in range(NS):
            for j in range(NS):
                k = c * NS * NS + i * NS + j
                a[k] = i * ROWPAD + c
                b[k] = j * ROWPAD + c
    return jnp.asarray(a), jnp.asarray(b)


def _topk_body(posT_tile_ref, pos3_ref, idx_ref, table_ref):
    b = pl.program_id(0)
    n_full = pos3_ref.shape[2]
    posT_t = posT_tile_ref[0]
    pos3 = pos3_ref[0]
    table_ref[...] = jnp.concatenate(
        [posT_t, jnp.zeros((posT_t.shape[0], ROWPAD - posT_t.shape[1]),
                           jnp.float32)], axis=1)
    sq_row = jnp.sum(pos3 * pos3, axis=0, keepdims=True)
    sq_col = jnp.sum(posT_t * posT_t, axis=1, keepdims=True)
    dot = jnp.dot(posT_t, pos3, preferred_element_type=jnp.float32)
    d = (sq_col + sq_row) - 2.0 * dot
    col = lax.broadcasted_iota(jnp.int32, d.shape, 1)
    work = d
    cols = []
    for _ in range(NS):
        idx_s = jnp.argmin(work, axis=1).astype(jnp.int32)[:, None]
        cols.append(idx_s)
        work = jnp.where(col == idx_s, jnp.float32(jnp.inf), work)
    idx_ref[...] = jnp.concatenate(cols, axis=1) + b * n_full


def _feat_body(g_ref, aidx_ref, bidx_ref, wq_ref, cvec_ref, out_ref):
    hi = jax.lax.Precision.HIGHEST
    G = g_ref[...]
    tile = G.shape[0]
    a_idx = jnp.broadcast_to(aidx_ref[...], (tile, 3 * NS * NS))
    b_idx = jnp.broadcast_to(bidx_ref[...], (tile, 3 * NS * NS))
    fa = jnp.take_along_axis(G, a_idx, axis=1,
                             mode="promise_in_bounds")
    fb = jnp.take_along_axis(G, b_idx, axis=1,
                             mode="promise_in_bounds")
    f = fa * fb
    out = jnp.dot(f, wq_ref[...], precision=hi,
                  preferred_element_type=jnp.float32)
    out = out + cvec_ref[...]
    out_ref[0] = out.T


def _sc_gather(table, idx_flat):
    n_idx = idx_flat.shape[0]
    info = plsc.get_sparse_core_info()
    nw = info.num_cores * info.num_subcores
    bpw = n_idx // nw
    mesh = plsc.VectorSubcoreMesh(core_axis_name="c", subcore_axis_name="s")

    @functools.partial(
        pl.kernel,
        out_type=jax.ShapeDtypeStruct((n_idx, ROWPAD), jnp.float32),
        mesh=mesh,
        compiler_params=pltpu.CompilerParams(use_tc_tiling_on_sc=False),
        scratch_types=[
            pltpu.VMEM((bpw,), jnp.int32),
            pltpu.VMEM((bpw, ROWPAD), jnp.float32),
            pltpu.SemaphoreType.DMA,
        ],
    )
    def gather_k(table_hbm, idx_hbm, out_hbm, idx_v, rows_v, sem):
        wid = lax.axis_index("s") * info.num_cores + lax.axis_index("c")
        base = wid * bpw
        pltpu.sync_copy(idx_hbm.at[pl.ds(base, bpw)], idx_v)
        pltpu.async_copy(table_hbm.at[idx_v], rows_v, sem).wait()
        pltpu.sync_copy(rows_v, out_hbm.at[pl.ds(base, bpw)])

    return gather_k(table, idx_flat)


def kernel(pos, W1, b1, W2, b2):
    B, C, N = pos.shape
    nt = N // TILE
    pos_t = jnp.transpose(pos, (0, 2, 1))

    idx, table = pl.pallas_call(
        _topk_body,
        grid=(B, nt),
        in_specs=[
            pl.BlockSpec((1, TILE, C), lambda b, t: (b, t, 0)),
            pl.BlockSpec((1, C, N), lambda b, t: (b, 0, 0)),
        ],
        out_specs=[
            pl.BlockSpec((TILE, NS), lambda b, t: (b * nt + t, 0)),
            pl.BlockSpec((TILE, ROWPAD), lambda b, t: (b * nt + t, 0)),
        ],
        out_shape=[
            jax.ShapeDtypeStruct((B * N, NS), jnp.int32),
            jax.ShapeDtypeStruct((B * N, ROWPAD), jnp.float32),
        ],
    )(pos_t, pos)

    rows = _sc_gather(table, idx.reshape(B * N * NS))
    G = rows.reshape(B * N, NS * ROWPAD)

    Wq, cvec = _fuse_weights(W1, b1, W2, b2)
    a_idx, b_idx = _selection_idx()
    mlp1 = W2.shape[0]

    out = pl.pallas_call(
        _feat_body,
        grid=(B, nt),
        in_specs=[
            pl.BlockSpec((TILE, NS * ROWPAD), lambda b, t: (b * nt + t, 0)),
            pl.BlockSpec((1, 3 * NS * NS), lambda b, t: (0, 0)),
            pl.BlockSpec((1, 3 * NS * NS), lambda b, t: (0, 0)),
            pl.BlockSpec((3 * NS * NS, mlp1), lambda b, t: (0, 0)),
            pl.BlockSpec((1, mlp1), lambda b, t: (0, 0)),
        ],
        out_specs=pl.BlockSpec((1, mlp1, TILE), lambda b, t: (b, 0, t)),
        out_shape=jax.ShapeDtypeStruct((B, mlp1, N), jnp.float32),
    )(G, a_idx[None, :], b_idx[None, :], Wq, cvec[None, :])
    return out
